# baseline (speedup 1.0000x reference)
import functools

import numpy as np

import jax
import jax.numpy as jnp
from jax import lax
from jax.experimental import pallas as pl
from jax.experimental.pallas import tpu as pltpu

F32 = jnp.float32
HIGHEST = lax.Precision.HIGHEST

_H = 256
_W = 256
_KW = 129
_NF = _H * _KW
_WP = 258
_LP = _WP * _WP
_MARG = 512
_CH = 512
_NCH = 131
_SPAN = _CH * _NCH
_X0W = 67968
_X1W = 68096
_OFFS = [dy * _WP + dx for dy in (-1, 0, 1) for dx in (-1, 0, 1)]
_WIN = 1280


def _np_dft_consts():
    w = np.arange(_W)[:, None].astype(np.float64)
    k = np.arange(_KW)[None, :].astype(np.float64)
    ang = 2.0 * np.pi / _W * (w * k)
    cw = np.cos(ang).astype(np.float32)
    sw = (-np.sin(ang)).astype(np.float32)
    ky = np.arange(_H)[:, None].astype(np.float64)
    h = np.arange(_H)[None, :].astype(np.float64)
    ang2 = 2.0 * np.pi / _H * (ky * h)
    ch = np.cos(ang2).astype(np.float32)
    sh = np.sin(ang2).astype(np.float32)
    cr = np.concatenate([ch, sh], axis=1)
    ci = np.concatenate([-sh, ch], axis=1)
    return cw, sw, cr, ci


_CW_NP, _SW_NP, _CR_NP, _CI_NP = _np_dft_consts()


def _np_mask():
    p = np.arange(_SPAN)
    row = p // _WP
    col = p % _WP
    valid = ((row >= 1) & (row <= _W) & (col >= 1) & (col <= _W)
             & (p < _LP))
    return valid.astype(np.float32)[None, :]


_MASK_NP = _np_mask()
_MASK2_NP = (_MASK_NP.reshape(_NCH, _CH) / 65536.0).astype(np.float32)


def _np_band_sel():
    s = np.zeros((3, 32, 32), np.float32)
    for d in range(3):
        s[d] = np.eye(32, k=1 - d, dtype=np.float32)
    return s


_S_NP = _np_band_sel()
_MAVG_NP = (np.kron(np.eye(32, dtype=np.float32),
                    np.ones((32, 1), np.float32)) / 32.0)


def _np_ef():
    ef = np.zeros((48, 48), np.float32)
    for r in range(48):
        i, k = r // 8, r % 8
        if i < 3:
            ef[r, 6 * k + i] = 1.0
        else:
            ef[r, 6 * k + 3 + (i - 3)] = 1.0
    return ef


_EF_NP = _np_ef()


def _np_selfconj():
    z = np.zeros((_H, _KW), np.float32)
    for r in (0, _H // 2):
        for c in (0, _W // 2):
            z[r, c] = 1.0
    return z


_SC_NP = _np_selfconj()


def _spec_body(x_ref, cw_ref, sw_ref, cr_ref, ci_ref, u_ref, sc_ref,
               mag_ref, msp_ref, psp_ref):
    inv_pi = np.float32(1.0 / np.pi)
    scb = sc_ref[...] > 0.5
    for c in range(3):
        xc = x_ref[0, c]
        xr = jnp.dot(xc, cw_ref[...], preferred_element_type=F32)
        xi = jnp.dot(xc, sw_ref[...], preferred_element_type=F32)
        p = jnp.concatenate([xr, xi], axis=0)
        yr = jnp.dot(cr_ref[...], p, preferred_element_type=F32)
        yi = jnp.dot(ci_ref[...], p, preferred_element_type=F32)
        yi = jnp.where(scb, 0.0, yi)
        m = jnp.clip(jnp.sqrt(yr * yr + yi * yi), 1e-8, 1e6)
        mag_ref[0, c] = m
        ml = jnp.clip(jnp.log1p(m), -20.0, 20.0)
        ph = jnp.clip(jnp.arctan2(yi, yr) * inv_pi, -1.0, 1.0)
        msp_ref[0, c] = jnp.dot(ml, u_ref[...], preferred_element_type=F32)
        psp_ref[0, c] = jnp.dot(ph, u_ref[...], preferred_element_type=F32)


def _conv_body(x0_ref, w1_ref, w2_ref, s1_ref, t1_ref, s2_ref, t2_ref,
               mask_ref, mask2_ref, out_ref, x1_ref):
    x1_ref[:, 0:_MARG] = jnp.zeros((96, _MARG), F32)
    x1_ref[:, _MARG + _SPAN:_X1W] = jnp.zeros((96, _X1W - _MARG - _SPAN), F32)
    w1c = w1_ref[...]
    w2c = w2_ref[...]
    s1 = s1_ref[...]
    t1 = t1_ref[...]
    s2 = s2_ref[...]
    t2 = t2_ref[...]

    def body1(i, c):
        bw = pl.multiple_of(128 + i * _CH, 128)
        xw = x0_ref[0, :, pl.ds(bw, _WIN)]
        p = jnp.concatenate(
            [xw[:, 384 + off:384 + off + _CH] for off in _OFFS], axis=0)
        acc = jnp.dot(w1c, p, preferred_element_type=F32)
        v = jnp.maximum(acc * s1 + t1, 0.0)
        v = v * mask_ref[:, pl.ds(i * _CH, _CH)]
        x1_ref[:, pl.ds(pl.multiple_of(_MARG + i * _CH, 128), _CH)] = v
        return c

    lax.fori_loop(0, _NCH, body1, 0)

    def body2(i, pool):
        bw = pl.multiple_of(i * _CH, 128)
        xw = x1_ref[:, pl.ds(bw, 1536)]
        p = jnp.concatenate(
            [xw[:, 126:894], xw[:, 384:1152], xw[:, 642:1410]], axis=0)
        h = lax.dot_general(p, w2c, (((0,), (0,)), ((), ())),
                            preferred_element_type=F32)
        a = (h[127:639, 0:128] + h[128:640, 128:256]
             + h[129:641, 256:384])
        v = jnp.maximum(a * s2 + t2, 0.0)
        m = mask2_ref[pl.ds(i, 1), :]
        return pool + jnp.dot(m, v, preferred_element_type=F32)

    pool = lax.fori_loop(0, _NCH, body2, jnp.zeros((1, 128), F32))
    out_ref[0] = pool[:, 0:96]


def _head_body(mag_ref, binb_ref, pool_ref, w1b_ref, w2b_ref, mavg_ref,
               ef_ref, rs1_ref, rt1_ref, rs2_ref, rt2_ref, rci_ref,
               aci_ref, acn_ref, ac1_ref, lw1_ref, lb1_ref, lng_ref,
               lnb_ref, lw2_ref, lb2_ref, out_ref):
    mag = mag_ref[...]
    mag96 = jnp.concatenate([mag, mag * mag], axis=0)
    d = jnp.dot(mag96, binb_ref[...], preferred_element_type=F32)
    rad = jnp.clip(d[0:48, 0:32] * rci_ref[...], 0.0, 1e6)
    rad16 = jnp.concatenate([rad[0:16], rad[16:32], rad[32:48]], axis=1)
    y1 = jnp.maximum(jnp.dot(rad16, w1b_ref[...],
                             preferred_element_type=F32)
                     * rs1_ref[...] + rt1_ref[...], 0.0)
    y2 = jnp.maximum(jnp.dot(y1, w2b_ref[...],
                             preferred_element_type=F32)
                     * rs2_ref[...] + rt2_ref[...], 0.0)
    rf = jnp.dot(y2, mavg_ref[...], preferred_element_type=F32)
    s1 = d[0:48, 32:40] * aci_ref[...]
    var = (d[48:96, 32:40] - acn_ref[...] * s1 * s1) * ac1_ref[...]
    std = jnp.sqrt(jnp.maximum(var, 0.0))
    ms = jnp.concatenate([s1[0:16], s1[16:32], s1[32:48],
                          std[0:16], std[16:32], std[32:48]], axis=1)
    az = jnp.dot(ms, ef_ref[...], preferred_element_type=F32)
    comb = jnp.concatenate([pool_ref[...], rf, az], axis=1)
    h = jnp.dot(comb, lw1_ref[...], preferred_element_type=F32) + lb1_ref[...]
    mu = jnp.mean(h, axis=-1, keepdims=True)
    dv = h - mu
    v2 = jnp.mean(dv * dv, axis=-1, keepdims=True)
    h = dv * lax.rsqrt(v2 + 1e-5) * lng_ref[...] + lnb_ref[...]
    h = jnp.maximum(h, 0.0)
    o = jnp.dot(h, lw2_ref[...], preferred_element_type=F32) + lb2_ref[...]
    out_ref[...] = jnp.clip(o, -100.0, 100.0)


def _bn_fold(bias, bnp):
    g, b, m, v = bnp
    s = g * lax.rsqrt(v + 1e-5)
    return s, (bias - m) * s + b


_CPARAMS = dict(dimension_semantics=("parallel", "arbitrary"),
                vmem_limit_bytes=56 * 1024 * 1024)


def kernel(x, mw1, mb1, mbn1, mw2, mb2, mbn2, pw1, pb1, pbn1, pw2, pb2, pbn2,
           rw1, rb1, rbn1, rw2, rb2, rbn2, lw1, lb1, lng, lnb, lw2, lb2):
    B = x.shape[0]

    cw = jnp.asarray(_CW_NP)
    sw = jnp.asarray(_SW_NP)
    cr = jnp.asarray(_CR_NP)
    ci = jnp.asarray(_CI_NP)
    u = jax.image.resize(jnp.eye(_KW, dtype=F32), (_KW, _W), 'bilinear')
    fy = jnp.fft.fftfreq(_H).astype(F32)
    fx = jnp.fft.rfftfreq(_W).astype(F32)
    fyg, fxg = jnp.meshgrid(fy, fx, indexing='ij')
    radius = jnp.sqrt(fxg ** 2 + fyg ** 2)
    rbins = jnp.clip((radius / (radius.max() + 1e-8) * 31).astype(jnp.int32),
                     0, 31).reshape(-1)
    angle = jnp.arctan2(fyg, fxg + 1e-8)
    abins = jnp.clip(((angle + np.pi) / (2 * np.pi) * 8).astype(jnp.int32),
                     0, 7).reshape(-1)
    r_oh = jax.nn.one_hot(rbins, 32, dtype=F32)
    a_oh = jax.nn.one_hot(abins, 8, dtype=F32)
    binb = jnp.concatenate([r_oh, a_oh], axis=1)
    rci = (1.0 / jnp.maximum(r_oh.sum(0), 1.0)).reshape(1, 32)
    a_cnt = a_oh.sum(0)
    aci = jnp.where(a_cnt > 0, 1.0 / a_cnt, 0.0).reshape(1, 8)
    acn = a_cnt.reshape(1, 8)
    ac1 = (1.0 / jnp.maximum(a_cnt - 1.0, 1.0)).reshape(1, 8)
    mask = jnp.asarray(_MASK_NP)
    ef = jnp.asarray(_EF_NP)
    mavg = jnp.asarray(_MAVG_NP)

    xc = jnp.clip(x.astype(F32), -10.0, 10.0)
    w1 = jnp.zeros((9, 96, 6), F32)
    w1 = w1.at[:, :64, :3].set(mw1.transpose(2, 3, 0, 1).reshape(9, 64, 3))
    w1 = w1.at[:, 64:, 3:].set(pw1.transpose(2, 3, 0, 1).reshape(9, 32, 3))
    w1cat = w1.transpose(1, 0, 2).reshape(96, 54)
    w2 = jnp.zeros((9, 96, 96), F32)
    w2 = w2.at[:, :64, :64].set(mw2.transpose(2, 3, 0, 1).reshape(9, 64, 64))
    w2 = w2.at[:, 64:, 64:].set(pw2.transpose(2, 3, 0, 1).reshape(9, 32, 32))
    wall = jnp.zeros((288, 384), F32)
    for g in range(3):
        for bb in range(3):
            wall = wall.at[96 * g:96 * g + 96, 128 * bb:128 * bb + 96].set(
                w2[g * 3 + bb].T)
    ms1, mt1 = _bn_fold(mb1, mbn1)
    ps1, pt1 = _bn_fold(pb1, pbn1)
    s1v = jnp.concatenate([ms1, ps1]).reshape(96, 1)
    t1v = jnp.concatenate([mt1, pt1]).reshape(96, 1)
    ms2, mt2 = _bn_fold(mb2, mbn2)
    ps2, pt2 = _bn_fold(pb2, pbn2)
    z32 = jnp.zeros((32,), F32)
    s2v = jnp.concatenate([ms2, ps2, z32]).reshape(1, 128)
    t2v = jnp.concatenate([mt2, pt2, z32]).reshape(1, 128)
    sband = jnp.asarray(_S_NP)
    w1b = jnp.einsum('ocd,dut->cuot', rw1, sband).reshape(96, 1024)
    w2b = jnp.einsum('ocd,dut->cuot', rw2, sband).reshape(1024, 1024)
    rs1_, rt1_ = _bn_fold(rb1, rbn1)
    rs2_, rt2_ = _bn_fold(rb2, rbn2)
    rs1 = jnp.repeat(rs1_, 32).reshape(1, 1024)
    rt1 = jnp.repeat(rt1_, 32).reshape(1, 1024)
    rs2 = jnp.repeat(rs2_, 32).reshape(1, 1024)
    rt2 = jnp.repeat(rt2_, 32).reshape(1, 1024)

    bcast2 = lambda c, j: (0, 0)
    bsel = lambda c, j: (c * (B // 2) + j, 0, 0, 0)
    mag, msp, psp = pl.pallas_call(
        _spec_body,
        grid=(2, B // 2),
        in_specs=[
            pl.BlockSpec((1, 3, _H, _W), bsel),
            pl.BlockSpec((_W, _KW), bcast2),
            pl.BlockSpec((_W, _KW), bcast2),
            pl.BlockSpec((_H, 2 * _H), bcast2),
            pl.BlockSpec((_H, 2 * _H), bcast2),
            pl.BlockSpec((_KW, _W), bcast2),
            pl.BlockSpec((_H, _KW), bcast2),
        ],
        out_specs=[
            pl.BlockSpec((1, 3, _H, _KW), bsel),
            pl.BlockSpec((1, 3, _H, _W), bsel),
            pl.BlockSpec((1, 3, _H, _W), bsel),
        ],
        out_shape=[
            jax.ShapeDtypeStruct((B, 3, _H, _KW), F32),
            jax.ShapeDtypeStruct((B, 3, _H, _W), F32),
            jax.ShapeDtypeStruct((B, 3, _H, _W), F32),
        ],
        compiler_params=pltpu.CompilerParams(**_CPARAMS),
        name="spectral_fft",
    )(xc, cw, sw, cr, ci, u, jnp.asarray(_SC_NP))

    sp = jnp.concatenate([msp, psp], axis=1)
    spp = jnp.pad(sp, ((0, 0), (0, 0), (1, 1), (1, 1)))
    x0 = spp.reshape(B, 6, _LP)
    x0e = jnp.pad(x0, ((0, 0), (0, 0), (_MARG, _X0W - _MARG - _LP)))
    mag48 = mag.transpose(1, 0, 2, 3).reshape(3 * B, _NF)

    bsel3 = lambda c, j: (c * (B // 2) + j, 0, 0)
    pool = pl.pallas_call(
        _conv_body,
        grid=(2, B // 2),
        in_specs=[
            pl.BlockSpec((1, 6, _X0W), bsel3),
            pl.BlockSpec((96, 54), bcast2),
            pl.BlockSpec((288, 384), bcast2),
            pl.BlockSpec((96, 1), bcast2),
            pl.BlockSpec((96, 1), bcast2),
            pl.BlockSpec((1, 128), bcast2),
            pl.BlockSpec((1, 128), bcast2),
            pl.BlockSpec((1, _SPAN), bcast2),
            pl.BlockSpec((_NCH, _CH), bcast2),
        ],
        out_specs=pl.BlockSpec((1, 1, 96), bsel3),
        out_shape=jax.ShapeDtypeStruct((B, 1, 96), F32),
        scratch_shapes=[pltpu.VMEM((96, _X1W), F32)],
        compiler_params=pltpu.CompilerParams(**_CPARAMS),
        name="conv_branches",
    )(x0e, w1cat, wall, s1v, t1v, s2v, t2v, mask, jnp.asarray(_MASK2_NP))
    poolt = pool.reshape(B, 96)

    out = pl.pallas_call(
        _head_body,
        out_shape=jax.ShapeDtypeStruct((B, 128), F32),
        compiler_params=pltpu.CompilerParams(
            vmem_limit_bytes=56 * 1024 * 1024),
        name="bin_head",
    )(mag48, binb, poolt, w1b, w2b, mavg, ef, rs1, rt1, rs2, rt2,
      rci, aci, acn, ac1, lw1.T, lb1.reshape(1, -1), lng.reshape(1, -1),
      lnb.reshape(1, -1), lw2.T, lb2.reshape(1, -1))
    return out


# speedup vs baseline: 1.8032x; 1.8032x over previous
import functools

import numpy as np

import jax
import jax.numpy as jnp
from jax import lax
from jax.experimental import pallas as pl
from jax.experimental.pallas import tpu as pltpu

F32 = jnp.float32
HIGHEST = lax.Precision.HIGHEST

_H = 256
_W = 256
_KW = 129
_NF = _H * _KW
_WP = 258
_LP = _WP * _WP
_MARG = 512
_CH = 512
_NCH = 131
_SPAN = _CH * _NCH
_X0W = 67968
_X1W = 67968
_OFFS = [dy * _WP + dx for dy in (-1, 0, 1) for dx in (-1, 0, 1)]
_WIN = 1280


def _np_dft_consts():
    w = np.arange(_W)[:, None].astype(np.float64)
    k = np.arange(_KW)[None, :].astype(np.float64)
    ang = 2.0 * np.pi / _W * (w * k)
    cw = np.cos(ang).astype(np.float32)
    sw = (-np.sin(ang)).astype(np.float32)
    ky = np.arange(_H)[:, None].astype(np.float64)
    h = np.arange(_H)[None, :].astype(np.float64)
    ang2 = 2.0 * np.pi / _H * (ky * h)
    ch = np.cos(ang2).astype(np.float32)
    sh = np.sin(ang2).astype(np.float32)
    cr = np.concatenate([ch, sh], axis=1)
    ci = np.concatenate([-sh, ch], axis=1)
    return cw, sw, cr, ci


_CW_NP, _SW_NP, _CR_NP, _CI_NP = _np_dft_consts()


def _np_mask():
    p = np.arange(_SPAN)
    row = p // _WP
    col = p % _WP
    valid = ((row >= 1) & (row <= _W) & (col >= 1) & (col <= _W)
             & (p < _LP))
    return valid.astype(np.float32)[None, :]


_MASK_NP = _np_mask()


def _np_band_sel():
    s = np.zeros((3, 32, 32), np.float32)
    for d in range(3):
        s[d] = np.eye(32, k=1 - d, dtype=np.float32)
    return s


_S_NP = _np_band_sel()
_MAVG_NP = (np.kron(np.eye(32, dtype=np.float32),
                    np.ones((32, 1), np.float32)) / 32.0)


def _np_ef():
    ef = np.zeros((48, 48), np.float32)
    for r in range(48):
        i, k = r // 8, r % 8
        if i < 3:
            ef[r, 6 * k + i] = 1.0
        else:
            ef[r, 6 * k + 3 + (i - 3)] = 1.0
    return ef


_EF_NP = _np_ef()


def _np_selfconj():
    z = np.zeros((_H, _KW), np.float32)
    for r in (0, _H // 2):
        for c in (0, _W // 2):
            z[r, c] = 1.0
    return z


_SC_NP = _np_selfconj()


def _spec_body(x_ref, cw_ref, sw_ref, cr_ref, ci_ref, u_ref, sc_ref,
               mag_ref, msp_ref, psp_ref):
    inv_pi = np.float32(1.0 / np.pi)
    scb = sc_ref[...] > 0.5
    for c in range(3):
        xc = x_ref[0, c]
        xr = jnp.dot(xc, cw_ref[...], preferred_element_type=F32)
        xi = jnp.dot(xc, sw_ref[...], preferred_element_type=F32)
        p = jnp.concatenate([xr, xi], axis=0)
        yr = jnp.dot(cr_ref[...], p, preferred_element_type=F32)
        yi = jnp.dot(ci_ref[...], p, preferred_element_type=F32)
        yi = jnp.where(scb, 0.0, yi)
        m = jnp.clip(jnp.sqrt(yr * yr + yi * yi), 1e-8, 1e6)
        mag_ref[0, c] = m
        ml = jnp.clip(jnp.log1p(m), -20.0, 20.0)
        ph = jnp.clip(jnp.arctan2(yi, yr) * inv_pi, -1.0, 1.0)
        msp_ref[0, c] = jnp.dot(ml, u_ref[...], preferred_element_type=F32)
        psp_ref[0, c] = jnp.dot(ph, u_ref[...], preferred_element_type=F32)


def _conv_body(x0_ref, w1_ref, w2_ref, s1_ref, t1_ref, s2_ref, t2_ref,
               mask_ref, out_ref, x1_ref):
    x1_ref[:, 0:_MARG] = jnp.zeros((96, _MARG), F32)
    x1_ref[:, _MARG + _SPAN:_X1W] = jnp.zeros((96, _X1W - _MARG - _SPAN), F32)
    w1c = w1_ref[...]
    w2c = w2_ref[...]
    s1 = s1_ref[...]
    t1 = t1_ref[...]
    s2 = s2_ref[...]
    t2 = t2_ref[...]

    def body1(i, c):
        bw = pl.multiple_of(128 + i * _CH, 128)
        xw = x0_ref[0, :, pl.ds(bw, _WIN)]
        p = jnp.concatenate(
            [xw[:, 384 + off:384 + off + _CH] for off in _OFFS], axis=0)
        acc = jnp.dot(w1c, p, preferred_element_type=F32)
        v = jnp.maximum(acc * s1 + t1, 0.0)
        v = v * mask_ref[:, pl.ds(i * _CH, _CH)]
        x1_ref[:, pl.ds(pl.multiple_of(_MARG + i * _CH, 128), _CH)] = v
        return c

    lax.fori_loop(0, _NCH, body1, 0, unroll=2)

    def body2(i, pool):
        bw = pl.multiple_of(128 + i * _CH, 128)
        xw = x1_ref[:, pl.ds(bw, _WIN)]
        p = jnp.concatenate(
            [xw[:, 384 + off:384 + off + _CH] for off in _OFFS], axis=0)
        acc = jnp.dot(w2c, p, preferred_element_type=F32)
        v = jnp.maximum(acc * s2 + t2, 0.0)
        v = v * mask_ref[:, pl.ds(i * _CH, _CH)]
        return pool + v

    pool = lax.fori_loop(0, _NCH, body2, jnp.zeros((96, _CH), F32), unroll=2)
    out_ref[0] = jnp.sum(pool, axis=1, keepdims=True) * np.float32(1.0 / 65536.0)


def _head_body(mag_ref, binb_ref, pool_ref, w1b_ref, w2b_ref, mavg_ref,
               ef_ref, rs1_ref, rt1_ref, rs2_ref, rt2_ref, rci_ref,
               aci_ref, acn_ref, ac1_ref, lw1_ref, lb1_ref, lng_ref,
               lnb_ref, lw2_ref, lb2_ref, out_ref):
    mag = mag_ref[...]
    mag96 = jnp.concatenate([mag, mag * mag], axis=0)
    d = jnp.dot(mag96, binb_ref[...], preferred_element_type=F32)
    rad = jnp.clip(d[0:48, 0:32] * rci_ref[...], 0.0, 1e6)
    rad16 = jnp.concatenate([rad[0:16], rad[16:32], rad[32:48]], axis=1)
    y1 = jnp.maximum(jnp.dot(rad16, w1b_ref[...],
                             preferred_element_type=F32)
                     * rs1_ref[...] + rt1_ref[...], 0.0)
    y2 = jnp.maximum(jnp.dot(y1, w2b_ref[...],
                             preferred_element_type=F32)
                     * rs2_ref[...] + rt2_ref[...], 0.0)
    rf = jnp.dot(y2, mavg_ref[...], preferred_element_type=F32)
    s1 = d[0:48, 32:40] * aci_ref[...]
    var = (d[48:96, 32:40] - acn_ref[...] * s1 * s1) * ac1_ref[...]
    std = jnp.sqrt(jnp.maximum(var, 0.0))
    ms = jnp.concatenate([s1[0:16], s1[16:32], s1[32:48],
                          std[0:16], std[16:32], std[32:48]], axis=1)
    az = jnp.dot(ms, ef_ref[...], preferred_element_type=F32)
    comb = jnp.concatenate([pool_ref[...], rf, az], axis=1)
    h = jnp.dot(comb, lw1_ref[...], preferred_element_type=F32) + lb1_ref[...]
    mu = jnp.mean(h, axis=-1, keepdims=True)
    dv = h - mu
    v2 = jnp.mean(dv * dv, axis=-1, keepdims=True)
    h = dv * lax.rsqrt(v2 + 1e-5) * lng_ref[...] + lnb_ref[...]
    h = jnp.maximum(h, 0.0)
    o = jnp.dot(h, lw2_ref[...], preferred_element_type=F32) + lb2_ref[...]
    out_ref[...] = jnp.clip(o, -100.0, 100.0)


def _bn_fold(bias, bnp):
    g, b, m, v = bnp
    s = g * lax.rsqrt(v + 1e-5)
    return s, (bias - m) * s + b


_CPARAMS = dict(dimension_semantics=("parallel", "arbitrary"),
                vmem_limit_bytes=56 * 1024 * 1024)


def kernel(x, mw1, mb1, mbn1, mw2, mb2, mbn2, pw1, pb1, pbn1, pw2, pb2, pbn2,
           rw1, rb1, rbn1, rw2, rb2, rbn2, lw1, lb1, lng, lnb, lw2, lb2):
    B = x.shape[0]

    cw = jnp.asarray(_CW_NP)
    sw = jnp.asarray(_SW_NP)
    cr = jnp.asarray(_CR_NP)
    ci = jnp.asarray(_CI_NP)
    u = jax.image.resize(jnp.eye(_KW, dtype=F32), (_KW, _W), 'bilinear')
    fy = jnp.fft.fftfreq(_H).astype(F32)
    fx = jnp.fft.rfftfreq(_W).astype(F32)
    fyg, fxg = jnp.meshgrid(fy, fx, indexing='ij')
    radius = jnp.sqrt(fxg ** 2 + fyg ** 2)
    rbins = jnp.clip((radius / (radius.max() + 1e-8) * 31).astype(jnp.int32),
                     0, 31).reshape(-1)
    angle = jnp.arctan2(fyg, fxg + 1e-8)
    abins = jnp.clip(((angle + np.pi) / (2 * np.pi) * 8).astype(jnp.int32),
                     0, 7).reshape(-1)
    r_oh = jax.nn.one_hot(rbins, 32, dtype=F32)
    a_oh = jax.nn.one_hot(abins, 8, dtype=F32)
    binb = jnp.concatenate([r_oh, a_oh], axis=1)
    rci = (1.0 / jnp.maximum(r_oh.sum(0), 1.0)).reshape(1, 32)
    a_cnt = a_oh.sum(0)
    aci = jnp.where(a_cnt > 0, 1.0 / a_cnt, 0.0).reshape(1, 8)
    acn = a_cnt.reshape(1, 8)
    ac1 = (1.0 / jnp.maximum(a_cnt - 1.0, 1.0)).reshape(1, 8)
    mask = jnp.asarray(_MASK_NP)
    ef = jnp.asarray(_EF_NP)
    mavg = jnp.asarray(_MAVG_NP)

    xc = jnp.clip(x.astype(F32), -10.0, 10.0)
    w1 = jnp.zeros((9, 96, 6), F32)
    w1 = w1.at[:, :64, :3].set(mw1.transpose(2, 3, 0, 1).reshape(9, 64, 3))
    w1 = w1.at[:, 64:, 3:].set(pw1.transpose(2, 3, 0, 1).reshape(9, 32, 3))
    w1cat = w1.transpose(1, 0, 2).reshape(96, 54)
    w2 = jnp.zeros((9, 96, 96), F32)
    w2 = w2.at[:, :64, :64].set(mw2.transpose(2, 3, 0, 1).reshape(9, 64, 64))
    w2 = w2.at[:, 64:, 64:].set(pw2.transpose(2, 3, 0, 1).reshape(9, 32, 32))
    w2cat = w2.transpose(1, 0, 2).reshape(96, 864)
    ms1, mt1 = _bn_fold(mb1, mbn1)
    ps1, pt1 = _bn_fold(pb1, pbn1)
    s1v = jnp.concatenate([ms1, ps1]).reshape(96, 1)
    t1v = jnp.concatenate([mt1, pt1]).reshape(96, 1)
    ms2, mt2 = _bn_fold(mb2, mbn2)
    ps2, pt2 = _bn_fold(pb2, pbn2)
    s2v = jnp.concatenate([ms2, ps2]).reshape(96, 1)
    t2v = jnp.concatenate([mt2, pt2]).reshape(96, 1)
    sband = jnp.asarray(_S_NP)
    w1b = jnp.einsum('ocd,dut->cuot', rw1, sband).reshape(96, 1024)
    w2b = jnp.einsum('ocd,dut->cuot', rw2, sband).reshape(1024, 1024)
    rs1_, rt1_ = _bn_fold(rb1, rbn1)
    rs2_, rt2_ = _bn_fold(rb2, rbn2)
    rs1 = jnp.repeat(rs1_, 32).reshape(1, 1024)
    rt1 = jnp.repeat(rt1_, 32).reshape(1, 1024)
    rs2 = jnp.repeat(rs2_, 32).reshape(1, 1024)
    rt2 = jnp.repeat(rt2_, 32).reshape(1, 1024)

    bcast2 = lambda c, j: (0, 0)
    bsel = lambda c, j: (c * (B // 2) + j, 0, 0, 0)
    mag, msp, psp = pl.pallas_call(
        _spec_body,
        grid=(2, B // 2),
        in_specs=[
            pl.BlockSpec((1, 3, _H, _W), bsel),
            pl.BlockSpec((_W, _KW), bcast2),
            pl.BlockSpec((_W, _KW), bcast2),
            pl.BlockSpec((_H, 2 * _H), bcast2),
            pl.BlockSpec((_H, 2 * _H), bcast2),
            pl.BlockSpec((_KW, _W), bcast2),
            pl.BlockSpec((_H, _KW), bcast2),
        ],
        out_specs=[
            pl.BlockSpec((1, 3, _H, _KW), bsel),
            pl.BlockSpec((1, 3, _H, _W), bsel),
            pl.BlockSpec((1, 3, _H, _W), bsel),
        ],
        out_shape=[
            jax.ShapeDtypeStruct((B, 3, _H, _KW), F32),
            jax.ShapeDtypeStruct((B, 3, _H, _W), F32),
            jax.ShapeDtypeStruct((B, 3, _H, _W), F32),
        ],
        compiler_params=pltpu.CompilerParams(**_CPARAMS),
        name="spectral_fft",
    )(xc, cw, sw, cr, ci, u, jnp.asarray(_SC_NP))

    sp = jnp.concatenate([msp, psp], axis=1)
    spp = jnp.pad(sp, ((0, 0), (0, 0), (1, 1), (1, 1)))
    x0 = spp.reshape(B, 6, _LP)
    x0e = jnp.pad(x0, ((0, 0), (0, 0), (_MARG, _X0W - _MARG - _LP)))
    mag48 = mag.transpose(1, 0, 2, 3).reshape(3 * B, _NF)

    bsel3 = lambda c, j: (c * (B // 2) + j, 0, 0)
    pool = pl.pallas_call(
        _conv_body,
        grid=(2, B // 2),
        in_specs=[
            pl.BlockSpec((1, 6, _X0W), bsel3),
            pl.BlockSpec((96, 54), bcast2),
            pl.BlockSpec((96, 864), bcast2),
            pl.BlockSpec((96, 1), bcast2),
            pl.BlockSpec((96, 1), bcast2),
            pl.BlockSpec((96, 1), bcast2),
            pl.BlockSpec((96, 1), bcast2),
            pl.BlockSpec((1, _SPAN), bcast2),
        ],
        out_specs=pl.BlockSpec((1, 96, 1), bsel3),
        out_shape=jax.ShapeDtypeStruct((B, 96, 1), F32),
        scratch_shapes=[pltpu.VMEM((96, _X1W), F32)],
        compiler_params=pltpu.CompilerParams(**_CPARAMS),
        name="conv_branches",
    )(x0e, w1cat, w2cat, s1v, t1v, s2v, t2v, mask)
    poolt = pool.reshape(B, 96)

    out = pl.pallas_call(
        _head_body,
        out_shape=jax.ShapeDtypeStruct((B, 128), F32),
        compiler_params=pltpu.CompilerParams(
            vmem_limit_bytes=56 * 1024 * 1024),
        name="bin_head",
    )(mag48, binb, poolt, w1b, w2b, mavg, ef, rs1, rt1, rs2, rt2,
      rci, aci, acn, ac1, lw1.T, lb1.reshape(1, -1), lng.reshape(1, -1),
      lnb.reshape(1, -1), lw2.T, lb2.reshape(1, -1))
    return out


# speedup vs baseline: 2.0504x; 1.1371x over previous
import functools

import numpy as np

import jax
import jax.numpy as jnp
from jax import lax
from jax.experimental import pallas as pl
from jax.experimental.pallas import tpu as pltpu

F32 = jnp.float32
HIGHEST = lax.Precision.HIGHEST

_H = 256
_W = 256
_KW = 129
_NF = _H * _KW
_WP = 258
_LP = _WP * _WP
_MARG = 512
_CH = 512
_NCH = 131
_SPAN = _CH * _NCH
_X0W = 67968
_X1W = 67968
_OFFS = [dy * _WP + dx for dy in (-1, 0, 1) for dx in (-1, 0, 1)]
_WIN = 1280


def _np_dft_consts():
    w = np.arange(_W)[:, None].astype(np.float64)
    k = np.arange(_KW)[None, :].astype(np.float64)
    ang = 2.0 * np.pi / _W * (w * k)
    cw = np.cos(ang).astype(np.float32)
    sw = (-np.sin(ang)).astype(np.float32)
    ky = np.arange(_H)[:, None].astype(np.float64)
    h = np.arange(_H)[None, :].astype(np.float64)
    ang2 = 2.0 * np.pi / _H * (ky * h)
    ch = np.cos(ang2).astype(np.float32)
    sh = np.sin(ang2).astype(np.float32)
    cr = np.concatenate([ch, sh], axis=1)
    ci = np.concatenate([-sh, ch], axis=1)
    return cw, sw, cr, ci


_CW_NP, _SW_NP, _CR_NP, _CI_NP = _np_dft_consts()


def _np_mask():
    p = np.arange(_SPAN)
    row = p // _WP
    col = p % _WP
    valid = ((row >= 1) & (row <= _W) & (col >= 1) & (col <= _W)
             & (p < _LP))
    return valid.astype(np.float32)[None, :]


_MASK_NP = _np_mask()


def _np_band_sel():
    s = np.zeros((3, 32, 32), np.float32)
    for d in range(3):
        s[d] = np.eye(32, k=1 - d, dtype=np.float32)
    return s


_S_NP = _np_band_sel()
_MAVG_NP = (np.kron(np.eye(32, dtype=np.float32),
                    np.ones((32, 1), np.float32)) / 32.0)


def _np_ef():
    ef = np.zeros((48, 48), np.float32)
    for r in range(48):
        i, k = r // 8, r % 8
        if i < 3:
            ef[r, 6 * k + i] = 1.0
        else:
            ef[r, 6 * k + 3 + (i - 3)] = 1.0
    return ef


_EF_NP = _np_ef()


def _np_selfconj():
    z = np.zeros((_H, _KW), np.float32)
    for r in (0, _H // 2):
        for c in (0, _W // 2):
            z[r, c] = 1.0
    return z


_SC_NP = _np_selfconj()


def _spec_body(x_ref, cw_ref, sw_ref, cr_ref, ci_ref, u_ref, sc_ref,
               mag_ref, msp_ref, psp_ref):
    inv_pi = np.float32(1.0 / np.pi)
    scb = sc_ref[...] > 0.5
    for c in range(3):
        xc = x_ref[0, c]
        xr = jnp.dot(xc, cw_ref[...], preferred_element_type=F32)
        xi = jnp.dot(xc, sw_ref[...], preferred_element_type=F32)
        p = jnp.concatenate([xr, xi], axis=0)
        yr = jnp.dot(cr_ref[...], p, preferred_element_type=F32)
        yi = jnp.dot(ci_ref[...], p, preferred_element_type=F32)
        yi = jnp.where(scb, 0.0, yi)
        m = jnp.clip(jnp.sqrt(yr * yr + yi * yi), 1e-8, 1e6)
        mag_ref[0, c] = m
        ml = jnp.clip(jnp.log1p(m), -20.0, 20.0)
        ph = jnp.clip(jnp.arctan2(yi, yr) * inv_pi, -1.0, 1.0)
        msp_ref[0, c] = jnp.dot(ml, u_ref[...], preferred_element_type=F32)
        psp_ref[0, c] = jnp.dot(ph, u_ref[...], preferred_element_type=F32)


def _conv_body(x0_ref, w1_ref, w2_ref, s1_ref, t1_ref, s2_ref, t2_ref,
               mask_ref, out_ref, x1_ref):
    x1_ref[:, 0:_MARG] = jnp.zeros((96, _MARG), F32)
    x1_ref[:, _MARG + _SPAN:_X1W] = jnp.zeros((96, _X1W - _MARG - _SPAN), F32)
    w1c = w1_ref[...]
    w2c = w2_ref[...]
    s1 = s1_ref[...]
    t1 = t1_ref[...]
    s2 = s2_ref[...]
    t2 = t2_ref[...]

    def body1(i, c):
        bw = pl.multiple_of(128 + i * _CH, 128)
        xw = x0_ref[0, :, pl.ds(bw, _WIN)]
        p = jnp.concatenate(
            [xw[:, 384 + off:384 + off + _CH] for off in _OFFS], axis=0)
        acc = jnp.dot(w1c, p, preferred_element_type=F32)
        v = jnp.maximum(acc * s1 + t1, 0.0)
        v = v * mask_ref[:, pl.ds(i * _CH, _CH)]
        x1_ref[:, pl.ds(pl.multiple_of(_MARG + i * _CH, 128), _CH)] = v
        return c

    lax.fori_loop(0, _NCH, body1, 0, unroll=4)

    def body2(i, pool):
        bw = pl.multiple_of(128 + i * _CH, 128)
        xw = x1_ref[:, pl.ds(bw, _WIN)]
        p = jnp.concatenate(
            [xw[:, 384 + off:384 + off + _CH] for off in _OFFS], axis=0)
        acc = jnp.dot(w2c, p, preferred_element_type=F32)
        v = jnp.maximum(acc * s2 + t2, 0.0)
        v = v * mask_ref[:, pl.ds(i * _CH, _CH)]
        return pool + v

    pool = lax.fori_loop(0, _NCH, body2, jnp.zeros((96, _CH), F32), unroll=4)
    out_ref[0] = jnp.sum(pool, axis=1, keepdims=True) * np.float32(1.0 / 65536.0)


def _head_body(mag_ref, binb_ref, pool_ref, w1b_ref, w2b_ref, mavg_ref,
               ef_ref, rs1_ref, rt1_ref, rs2_ref, rt2_ref, rci_ref,
               aci_ref, acn_ref, ac1_ref, lw1_ref, lb1_ref, lng_ref,
               lnb_ref, lw2_ref, lb2_ref, out_ref):
    mag = mag_ref[...]
    mag96 = jnp.concatenate([mag, mag * mag], axis=0)
    d = jnp.dot(mag96, binb_ref[...], preferred_element_type=F32)
    rad = jnp.clip(d[0:48, 0:32] * rci_ref[...], 0.0, 1e6)
    rad16 = jnp.concatenate([rad[0:16], rad[16:32], rad[32:48]], axis=1)
    y1 = jnp.maximum(jnp.dot(rad16, w1b_ref[...],
                             preferred_element_type=F32)
                     * rs1_ref[...] + rt1_ref[...], 0.0)
    y2 = jnp.maximum(jnp.dot(y1, w2b_ref[...],
                             preferred_element_type=F32)
                     * rs2_ref[...] + rt2_ref[...], 0.0)
    rf = jnp.dot(y2, mavg_ref[...], preferred_element_type=F32)
    s1 = d[0:48, 32:40] * aci_ref[...]
    var = (d[48:96, 32:40] - acn_ref[...] * s1 * s1) * ac1_ref[...]
    std = jnp.sqrt(jnp.maximum(var, 0.0))
    ms = jnp.concatenate([s1[0:16], s1[16:32], s1[32:48],
                          std[0:16], std[16:32], std[32:48]], axis=1)
    az = jnp.dot(ms, ef_ref[...], preferred_element_type=F32)
    comb = jnp.concatenate([pool_ref[...], rf, az], axis=1)
    h = jnp.dot(comb, lw1_ref[...], preferred_element_type=F32) + lb1_ref[...]
    mu = jnp.mean(h, axis=-1, keepdims=True)
    dv = h - mu
    v2 = jnp.mean(dv * dv, axis=-1, keepdims=True)
    h = dv * lax.rsqrt(v2 + 1e-5) * lng_ref[...] + lnb_ref[...]
    h = jnp.maximum(h, 0.0)
    o = jnp.dot(h, lw2_ref[...], preferred_element_type=F32) + lb2_ref[...]
    out_ref[...] = jnp.clip(o, -100.0, 100.0)


def _bn_fold(bias, bnp):
    g, b, m, v = bnp
    s = g * lax.rsqrt(v + 1e-5)
    return s, (bias - m) * s + b


_CPARAMS = dict(dimension_semantics=("parallel", "arbitrary"),
                vmem_limit_bytes=56 * 1024 * 1024)


def kernel(x, mw1, mb1, mbn1, mw2, mb2, mbn2, pw1, pb1, pbn1, pw2, pb2, pbn2,
           rw1, rb1, rbn1, rw2, rb2, rbn2, lw1, lb1, lng, lnb, lw2, lb2):
    B = x.shape[0]

    cw = jnp.asarray(_CW_NP)
    sw = jnp.asarray(_SW_NP)
    cr = jnp.asarray(_CR_NP)
    ci = jnp.asarray(_CI_NP)
    u = jax.image.resize(jnp.eye(_KW, dtype=F32), (_KW, _W), 'bilinear')
    fy = jnp.fft.fftfreq(_H).astype(F32)
    fx = jnp.fft.rfftfreq(_W).astype(F32)
    fyg, fxg = jnp.meshgrid(fy, fx, indexing='ij')
    radius = jnp.sqrt(fxg ** 2 + fyg ** 2)
    rbins = jnp.clip((radius / (radius.max() + 1e-8) * 31).astype(jnp.int32),
                     0, 31).reshape(-1)
    angle = jnp.arctan2(fyg, fxg + 1e-8)
    abins = jnp.clip(((angle + np.pi) / (2 * np.pi) * 8).astype(jnp.int32),
                     0, 7).reshape(-1)
    r_oh = jax.nn.one_hot(rbins, 32, dtype=F32)
    a_oh = jax.nn.one_hot(abins, 8, dtype=F32)
    binb = jnp.concatenate([r_oh, a_oh], axis=1)
    rci = (1.0 / jnp.maximum(r_oh.sum(0), 1.0)).reshape(1, 32)
    a_cnt = a_oh.sum(0)
    aci = jnp.where(a_cnt > 0, 1.0 / a_cnt, 0.0).reshape(1, 8)
    acn = a_cnt.reshape(1, 8)
    ac1 = (1.0 / jnp.maximum(a_cnt - 1.0, 1.0)).reshape(1, 8)
    mask = jnp.asarray(_MASK_NP)
    ef = jnp.asarray(_EF_NP)
    mavg = jnp.asarray(_MAVG_NP)

    xc = jnp.clip(x.astype(F32), -10.0, 10.0)
    w1 = jnp.zeros((9, 96, 6), F32)
    w1 = w1.at[:, :64, :3].set(mw1.transpose(2, 3, 0, 1).reshape(9, 64, 3))
    w1 = w1.at[:, 64:, 3:].set(pw1.transpose(2, 3, 0, 1).reshape(9, 32, 3))
    w1cat = w1.transpose(1, 0, 2).reshape(96, 54)
    w2 = jnp.zeros((9, 96, 96), F32)
    w2 = w2.at[:, :64, :64].set(mw2.transpose(2, 3, 0, 1).reshape(9, 64, 64))
    w2 = w2.at[:, 64:, 64:].set(pw2.transpose(2, 3, 0, 1).reshape(9, 32, 32))
    w2cat = w2.transpose(1, 0, 2).reshape(96, 864)
    ms1, mt1 = _bn_fold(mb1, mbn1)
    ps1, pt1 = _bn_fold(pb1, pbn1)
    s1v = jnp.concatenate([ms1, ps1]).reshape(96, 1)
    t1v = jnp.concatenate([mt1, pt1]).reshape(96, 1)
    ms2, mt2 = _bn_fold(mb2, mbn2)
    ps2, pt2 = _bn_fold(pb2, pbn2)
    s2v = jnp.concatenate([ms2, ps2]).reshape(96, 1)
    t2v = jnp.concatenate([mt2, pt2]).reshape(96, 1)
    sband = jnp.asarray(_S_NP)
    w1b = jnp.einsum('ocd,dut->cuot', rw1, sband).reshape(96, 1024)
    w2b = jnp.einsum('ocd,dut->cuot', rw2, sband).reshape(1024, 1024)
    rs1_, rt1_ = _bn_fold(rb1, rbn1)
    rs2_, rt2_ = _bn_fold(rb2, rbn2)
    rs1 = jnp.repeat(rs1_, 32).reshape(1, 1024)
    rt1 = jnp.repeat(rt1_, 32).reshape(1, 1024)
    rs2 = jnp.repeat(rs2_, 32).reshape(1, 1024)
    rt2 = jnp.repeat(rt2_, 32).reshape(1, 1024)

    bcast2 = lambda c, j: (0, 0)
    bsel = lambda c, j: (c * (B // 2) + j, 0, 0, 0)
    mag, msp, psp = pl.pallas_call(
        _spec_body,
        grid=(2, B // 2),
        in_specs=[
            pl.BlockSpec((1, 3, _H, _W), bsel),
            pl.BlockSpec((_W, _KW), bcast2),
            pl.BlockSpec((_W, _KW), bcast2),
            pl.BlockSpec((_H, 2 * _H), bcast2),
            pl.BlockSpec((_H, 2 * _H), bcast2),
            pl.BlockSpec((_KW, _W), bcast2),
            pl.BlockSpec((_H, _KW), bcast2),
        ],
        out_specs=[
            pl.BlockSpec((1, 3, _H, _KW), bsel),
            pl.BlockSpec((1, 3, _H, _W), bsel),
            pl.BlockSpec((1, 3, _H, _W), bsel),
        ],
        out_shape=[
            jax.ShapeDtypeStruct((B, 3, _H, _KW), F32),
            jax.ShapeDtypeStruct((B, 3, _H, _W), F32),
            jax.ShapeDtypeStruct((B, 3, _H, _W), F32),
        ],
        compiler_params=pltpu.CompilerParams(**_CPARAMS),
        name="spectral_fft",
    )(xc, cw, sw, cr, ci, u, jnp.asarray(_SC_NP))

    sp = jnp.concatenate([msp, psp], axis=1)
    spp = jnp.pad(sp, ((0, 0), (0, 0), (1, 1), (1, 1)))
    x0 = spp.reshape(B, 6, _LP)
    x0e = jnp.pad(x0, ((0, 0), (0, 0), (_MARG, _X0W - _MARG - _LP)))
    mag48 = mag.transpose(1, 0, 2, 3).reshape(3 * B, _NF)

    bsel3 = lambda c, j: (c * (B // 2) + j, 0, 0)
    pool = pl.pallas_call(
        _conv_body,
        grid=(2, B // 2),
        in_specs=[
            pl.BlockSpec((1, 6, _X0W), bsel3),
            pl.BlockSpec((96, 54), bcast2),
            pl.BlockSpec((96, 864), bcast2),
            pl.BlockSpec((96, 1), bcast2),
            pl.BlockSpec((96, 1), bcast2),
            pl.BlockSpec((96, 1), bcast2),
            pl.BlockSpec((96, 1), bcast2),
            pl.BlockSpec((1, _SPAN), bcast2),
        ],
        out_specs=pl.BlockSpec((1, 96, 1), bsel3),
        out_shape=jax.ShapeDtypeStruct((B, 96, 1), F32),
        scratch_shapes=[pltpu.VMEM((96, _X1W), F32)],
        compiler_params=pltpu.CompilerParams(**_CPARAMS),
        name="conv_branches",
    )(x0e, w1cat, w2cat, s1v, t1v, s2v, t2v, mask)
    poolt = pool.reshape(B, 96)

    out = pl.pallas_call(
        _head_body,
        out_shape=jax.ShapeDtypeStruct((B, 128), F32),
        compiler_params=pltpu.CompilerParams(
            vmem_limit_bytes=56 * 1024 * 1024),
        name="bin_head",
    )(mag48, binb, poolt, w1b, w2b, mavg, ef, rs1, rt1, rs2, rt2,
      rci, aci, acn, ac1, lw1.T, lb1.reshape(1, -1), lng.reshape(1, -1),
      lnb.reshape(1, -1), lw2.T, lb2.reshape(1, -1))
    return out


# speedup vs baseline: 2.1374x; 1.0424x over previous
import functools

import numpy as np

import jax
import jax.numpy as jnp
from jax import lax
from jax.experimental import pallas as pl
from jax.experimental.pallas import tpu as pltpu

F32 = jnp.float32
HIGHEST = lax.Precision.HIGHEST

_H = 256
_W = 256
_KW = 129
_NF = _H * _KW
_WP = 258
_LP = _WP * _WP
_MARG = 512
_CH = 512
_NCH = 131
_SPAN = _CH * _NCH
_X0W = 67968
_X1W = 67968
_OFFS = [dy * _WP + dx for dy in (-1, 0, 1) for dx in (-1, 0, 1)]
_WIN = 1280


def _np_dft_consts():
    w = np.arange(_W)[:, None].astype(np.float64)
    k = np.arange(_KW)[None, :].astype(np.float64)
    ang = 2.0 * np.pi / _W * (w * k)
    cw = np.cos(ang).astype(np.float32)
    sw = (-np.sin(ang)).astype(np.float32)
    ky = np.arange(_H)[:, None].astype(np.float64)
    h = np.arange(_H)[None, :].astype(np.float64)
    ang2 = 2.0 * np.pi / _H * (ky * h)
    ch = np.cos(ang2).astype(np.float32)
    sh = np.sin(ang2).astype(np.float32)
    cr = np.concatenate([ch, sh], axis=1)
    ci = np.concatenate([-sh, ch], axis=1)
    return cw, sw, cr, ci


_CW_NP, _SW_NP, _CR_NP, _CI_NP = _np_dft_consts()


def _np_mask():
    p = np.arange(_SPAN)
    row = p // _WP
    col = p % _WP
    valid = ((row >= 1) & (row <= _W) & (col >= 1) & (col <= _W)
             & (p < _LP))
    return valid.astype(np.float32)[None, :]


_MASK_NP = _np_mask()


def _np_band_sel():
    s = np.zeros((3, 32, 32), np.float32)
    for d in range(3):
        s[d] = np.eye(32, k=1 - d, dtype=np.float32)
    return s


_S_NP = _np_band_sel()
_MAVG_NP = (np.kron(np.eye(32, dtype=np.float32),
                    np.ones((32, 1), np.float32)) / 32.0)


def _np_ef():
    ef = np.zeros((48, 48), np.float32)
    for r in range(48):
        i, k = r // 8, r % 8
        if i < 3:
            ef[r, 6 * k + i] = 1.0
        else:
            ef[r, 6 * k + 3 + (i - 3)] = 1.0
    return ef


_EF_NP = _np_ef()


def _np_selfconj():
    z = np.zeros((_H, _KW), np.float32)
    for r in (0, _H // 2):
        for c in (0, _W // 2):
            z[r, c] = 1.0
    return z


_SC_NP = _np_selfconj()


def _spec_body(x_ref, cw_ref, sw_ref, cr_ref, ci_ref, u_ref, sc_ref,
               mag_ref, msp_ref, psp_ref):
    inv_pi = np.float32(1.0 / np.pi)
    scb = sc_ref[...] > 0.5
    for c in range(3):
        xc = x_ref[0, c]
        xr = jnp.dot(xc, cw_ref[...], preferred_element_type=F32)
        xi = jnp.dot(xc, sw_ref[...], preferred_element_type=F32)
        p = jnp.concatenate([xr, xi], axis=0)
        yr = jnp.dot(cr_ref[...], p, preferred_element_type=F32)
        yi = jnp.dot(ci_ref[...], p, preferred_element_type=F32)
        yi = jnp.where(scb, 0.0, yi)
        m = jnp.clip(jnp.sqrt(yr * yr + yi * yi), 1e-8, 1e6)
        mag_ref[0, c] = m
        ml = jnp.clip(jnp.log1p(m), -20.0, 20.0)
        ph = jnp.clip(jnp.arctan2(yi, yr) * inv_pi, -1.0, 1.0)
        msp_ref[0, c] = jnp.dot(ml, u_ref[...], preferred_element_type=F32)
        psp_ref[0, c] = jnp.dot(ph, u_ref[...], preferred_element_type=F32)


def _conv_body(x0_ref, w1_ref, w2_ref, s1_ref, t1_ref, s2_ref, t2_ref,
               mask_ref, out_ref, x1_ref):
    x1_ref[:, 0:_MARG] = jnp.zeros((96, _MARG), F32)
    x1_ref[:, _MARG + _SPAN:_X1W] = jnp.zeros((96, _X1W - _MARG - _SPAN), F32)
    w1c = w1_ref[...]
    w2c = w2_ref[...]
    s1 = s1_ref[...]
    t1 = t1_ref[...]
    s2 = s2_ref[...]
    t2 = t2_ref[...]

    def body1(i, c):
        bw = pl.multiple_of(128 + i * _CH, 128)
        xw = x0_ref[0, :, pl.ds(bw, _WIN)]
        p = jnp.concatenate(
            [xw[:, 384 + off:384 + off + _CH] for off in _OFFS], axis=0)
        acc = jnp.dot(w1c, p, preferred_element_type=F32)
        v = jnp.maximum(acc * s1 + t1, 0.0)
        v = v * mask_ref[:, pl.ds(i * _CH, _CH)]
        x1_ref[:, pl.ds(pl.multiple_of(_MARG + i * _CH, 128), _CH)] = v
        return c

    lax.fori_loop(0, _NCH, body1, 0, unroll=8)

    def body2(i, pool):
        bw = pl.multiple_of(128 + i * _CH, 128)
        xw = x1_ref[:, pl.ds(bw, _WIN)]
        p = jnp.concatenate(
            [xw[:, 384 + off:384 + off + _CH] for off in _OFFS], axis=0)
        acc = jnp.dot(w2c, p, preferred_element_type=F32)
        v = jnp.maximum(acc * s2 + t2, 0.0)
        v = v * mask_ref[:, pl.ds(i * _CH, _CH)]
        return pool + v

    pool = lax.fori_loop(0, _NCH, body2, jnp.zeros((96, _CH), F32), unroll=8)
    out_ref[0] = jnp.sum(pool, axis=1, keepdims=True) * np.float32(1.0 / 65536.0)


def _head_body(mag_ref, binb_ref, pool_ref, w1b_ref, w2b_ref, mavg_ref,
               ef_ref, rs1_ref, rt1_ref, rs2_ref, rt2_ref, rci_ref,
               aci_ref, acn_ref, ac1_ref, lw1_ref, lb1_ref, lng_ref,
               lnb_ref, lw2_ref, lb2_ref, out_ref):
    mag = mag_ref[...]
    mag96 = jnp.concatenate([mag, mag * mag], axis=0)
    d = jnp.dot(mag96, binb_ref[...], preferred_element_type=F32)
    rad = jnp.clip(d[0:48, 0:32] * rci_ref[...], 0.0, 1e6)
    rad16 = jnp.concatenate([rad[0:16], rad[16:32], rad[32:48]], axis=1)
    y1 = jnp.maximum(jnp.dot(rad16, w1b_ref[...],
                             preferred_element_type=F32)
                     * rs1_ref[...] + rt1_ref[...], 0.0)
    y2 = jnp.maximum(jnp.dot(y1, w2b_ref[...],
                             preferred_element_type=F32)
                     * rs2_ref[...] + rt2_ref[...], 0.0)
    rf = jnp.dot(y2, mavg_ref[...], preferred_element_type=F32)
    s1 = d[0:48, 32:40] * aci_ref[...]
    var = (d[48:96, 32:40] - acn_ref[...] * s1 * s1) * ac1_ref[...]
    std = jnp.sqrt(jnp.maximum(var, 0.0))
    ms = jnp.concatenate([s1[0:16], s1[16:32], s1[32:48],
                          std[0:16], std[16:32], std[32:48]], axis=1)
    az = jnp.dot(ms, ef_ref[...], preferred_element_type=F32)
    comb = jnp.concatenate([pool_ref[...], rf, az], axis=1)
    h = jnp.dot(comb, lw1_ref[...], preferred_element_type=F32) + lb1_ref[...]
    mu = jnp.mean(h, axis=-1, keepdims=True)
    dv = h - mu
    v2 = jnp.mean(dv * dv, axis=-1, keepdims=True)
    h = dv * lax.rsqrt(v2 + 1e-5) * lng_ref[...] + lnb_ref[...]
    h = jnp.maximum(h, 0.0)
    o = jnp.dot(h, lw2_ref[...], preferred_element_type=F32) + lb2_ref[...]
    out_ref[...] = jnp.clip(o, -100.0, 100.0)


def _bn_fold(bias, bnp):
    g, b, m, v = bnp
    s = g * lax.rsqrt(v + 1e-5)
    return s, (bias - m) * s + b


_CPARAMS = dict(dimension_semantics=("parallel", "arbitrary"),
                vmem_limit_bytes=56 * 1024 * 1024)


def kernel(x, mw1, mb1, mbn1, mw2, mb2, mbn2, pw1, pb1, pbn1, pw2, pb2, pbn2,
           rw1, rb1, rbn1, rw2, rb2, rbn2, lw1, lb1, lng, lnb, lw2, lb2):
    B = x.shape[0]

    cw = jnp.asarray(_CW_NP)
    sw = jnp.asarray(_SW_NP)
    cr = jnp.asarray(_CR_NP)
    ci = jnp.asarray(_CI_NP)
    u = jax.image.resize(jnp.eye(_KW, dtype=F32), (_KW, _W), 'bilinear')
    fy = jnp.fft.fftfreq(_H).astype(F32)
    fx = jnp.fft.rfftfreq(_W).astype(F32)
    fyg, fxg = jnp.meshgrid(fy, fx, indexing='ij')
    radius = jnp.sqrt(fxg ** 2 + fyg ** 2)
    rbins = jnp.clip((radius / (radius.max() + 1e-8) * 31).astype(jnp.int32),
                     0, 31).reshape(-1)
    angle = jnp.arctan2(fyg, fxg + 1e-8)
    abins = jnp.clip(((angle + np.pi) / (2 * np.pi) * 8).astype(jnp.int32),
                     0, 7).reshape(-1)
    r_oh = jax.nn.one_hot(rbins, 32, dtype=F32)
    a_oh = jax.nn.one_hot(abins, 8, dtype=F32)
    binb = jnp.concatenate([r_oh, a_oh], axis=1)
    rci = (1.0 / jnp.maximum(r_oh.sum(0), 1.0)).reshape(1, 32)
    a_cnt = a_oh.sum(0)
    aci = jnp.where(a_cnt > 0, 1.0 / a_cnt, 0.0).reshape(1, 8)
    acn = a_cnt.reshape(1, 8)
    ac1 = (1.0 / jnp.maximum(a_cnt - 1.0, 1.0)).reshape(1, 8)
    mask = jnp.asarray(_MASK_NP)
    ef = jnp.asarray(_EF_NP)
    mavg = jnp.asarray(_MAVG_NP)

    xc = jnp.clip(x.astype(F32), -10.0, 10.0)
    w1 = jnp.zeros((9, 96, 6), F32)
    w1 = w1.at[:, :64, :3].set(mw1.transpose(2, 3, 0, 1).reshape(9, 64, 3))
    w1 = w1.at[:, 64:, 3:].set(pw1.transpose(2, 3, 0, 1).reshape(9, 32, 3))
    w1cat = w1.transpose(1, 0, 2).reshape(96, 54)
    w2 = jnp.zeros((9, 96, 96), F32)
    w2 = w2.at[:, :64, :64].set(mw2.transpose(2, 3, 0, 1).reshape(9, 64, 64))
    w2 = w2.at[:, 64:, 64:].set(pw2.transpose(2, 3, 0, 1).reshape(9, 32, 32))
    w2cat = w2.transpose(1, 0, 2).reshape(96, 864)
    ms1, mt1 = _bn_fold(mb1, mbn1)
    ps1, pt1 = _bn_fold(pb1, pbn1)
    s1v = jnp.concatenate([ms1, ps1]).reshape(96, 1)
    t1v = jnp.concatenate([mt1, pt1]).reshape(96, 1)
    ms2, mt2 = _bn_fold(mb2, mbn2)
    ps2, pt2 = _bn_fold(pb2, pbn2)
    s2v = jnp.concatenate([ms2, ps2]).reshape(96, 1)
    t2v = jnp.concatenate([mt2, pt2]).reshape(96, 1)
    sband = jnp.asarray(_S_NP)
    w1b = jnp.einsum('ocd,dut->cuot', rw1, sband).reshape(96, 1024)
    w2b = jnp.einsum('ocd,dut->cuot', rw2, sband).reshape(1024, 1024)
    rs1_, rt1_ = _bn_fold(rb1, rbn1)
    rs2_, rt2_ = _bn_fold(rb2, rbn2)
    rs1 = jnp.repeat(rs1_, 32).reshape(1, 1024)
    rt1 = jnp.repeat(rt1_, 32).reshape(1, 1024)
    rs2 = jnp.repeat(rs2_, 32).reshape(1, 1024)
    rt2 = jnp.repeat(rt2_, 32).reshape(1, 1024)

    bcast2 = lambda c, j: (0, 0)
    bsel = lambda c, j: (c * (B // 2) + j, 0, 0, 0)
    mag, msp, psp = pl.pallas_call(
        _spec_body,
        grid=(2, B // 2),
        in_specs=[
            pl.BlockSpec((1, 3, _H, _W), bsel),
            pl.BlockSpec((_W, _KW), bcast2),
            pl.BlockSpec((_W, _KW), bcast2),
            pl.BlockSpec((_H, 2 * _H), bcast2),
            pl.BlockSpec((_H, 2 * _H), bcast2),
            pl.BlockSpec((_KW, _W), bcast2),
            pl.BlockSpec((_H, _KW), bcast2),
        ],
        out_specs=[
            pl.BlockSpec((1, 3, _H, _KW), bsel),
            pl.BlockSpec((1, 3, _H, _W), bsel),
            pl.BlockSpec((1, 3, _H, _W), bsel),
        ],
        out_shape=[
            jax.ShapeDtypeStruct((B, 3, _H, _KW), F32),
            jax.ShapeDtypeStruct((B, 3, _H, _W), F32),
            jax.ShapeDtypeStruct((B, 3, _H, _W), F32),
        ],
        compiler_params=pltpu.CompilerParams(**_CPARAMS),
        name="spectral_fft",
    )(xc, cw, sw, cr, ci, u, jnp.asarray(_SC_NP))

    sp = jnp.concatenate([msp, psp], axis=1)
    spp = jnp.pad(sp, ((0, 0), (0, 0), (1, 1), (1, 1)))
    x0 = spp.reshape(B, 6, _LP)
    x0e = jnp.pad(x0, ((0, 0), (0, 0), (_MARG, _X0W - _MARG - _LP)))
    mag48 = mag.transpose(1, 0, 2, 3).reshape(3 * B, _NF)

    bsel3 = lambda c, j: (c * (B // 2) + j, 0, 0)
    pool = pl.pallas_call(
        _conv_body,
        grid=(2, B // 2),
        in_specs=[
            pl.BlockSpec((1, 6, _X0W), bsel3),
            pl.BlockSpec((96, 54), bcast2),
            pl.BlockSpec((96, 864), bcast2),
            pl.BlockSpec((96, 1), bcast2),
            pl.BlockSpec((96, 1), bcast2),
            pl.BlockSpec((96, 1), bcast2),
            pl.BlockSpec((96, 1), bcast2),
            pl.BlockSpec((1, _SPAN), bcast2),
        ],
        out_specs=pl.BlockSpec((1, 96, 1), bsel3),
        out_shape=jax.ShapeDtypeStruct((B, 96, 1), F32),
        scratch_shapes=[pltpu.VMEM((96, _X1W), F32)],
        compiler_params=pltpu.CompilerParams(**_CPARAMS),
        name="conv_branches",
    )(x0e, w1cat, w2cat, s1v, t1v, s2v, t2v, mask)
    poolt = pool.reshape(B, 96)

    out = pl.pallas_call(
        _head_body,
        out_shape=jax.ShapeDtypeStruct((B, 128), F32),
        compiler_params=pltpu.CompilerParams(
            vmem_limit_bytes=56 * 1024 * 1024),
        name="bin_head",
    )(mag48, binb, poolt, w1b, w2b, mavg, ef, rs1, rt1, rs2, rt2,
      rci, aci, acn, ac1, lw1.T, lb1.reshape(1, -1), lng.reshape(1, -1),
      lnb.reshape(1, -1), lw2.T, lb2.reshape(1, -1))
    return out


# speedup vs baseline: 2.1908x; 1.0250x over previous
import functools

import numpy as np

import jax
import jax.numpy as jnp
from jax import lax
from jax.experimental import pallas as pl
from jax.experimental.pallas import tpu as pltpu

F32 = jnp.float32
HIGHEST = lax.Precision.HIGHEST

_H = 256
_W = 256
_KW = 129
_NF = _H * _KW
_WP = 258
_LP = _WP * _WP
_MARG = 512
_CH = 512
_NCH = 131
_SPAN = _CH * _NCH
_X0W = 67968
_X1W = 67968
_OFFS = [dy * _WP + dx for dy in (-1, 0, 1) for dx in (-1, 0, 1)]
_WIN = 1280


def _np_dft_consts():
    w = np.arange(_W)[:, None].astype(np.float64)
    k = np.arange(_KW)[None, :].astype(np.float64)
    ang = 2.0 * np.pi / _W * (w * k)
    cw = np.cos(ang).astype(np.float32)
    sw = (-np.sin(ang)).astype(np.float32)
    ky = np.arange(_H)[:, None].astype(np.float64)
    h = np.arange(_H)[None, :].astype(np.float64)
    ang2 = 2.0 * np.pi / _H * (ky * h)
    ch = np.cos(ang2).astype(np.float32)
    sh = np.sin(ang2).astype(np.float32)
    cr = np.concatenate([ch, sh], axis=1)
    ci = np.concatenate([-sh, ch], axis=1)
    return cw, sw, cr, ci


_CW_NP, _SW_NP, _CR_NP, _CI_NP = _np_dft_consts()


def _np_mask():
    p = np.arange(_SPAN)
    row = p // _WP
    col = p % _WP
    valid = ((row >= 1) & (row <= _W) & (col >= 1) & (col <= _W)
             & (p < _LP))
    return valid.astype(np.float32)[None, :]


_MASK_NP = _np_mask()


def _np_band_sel():
    s = np.zeros((3, 32, 32), np.float32)
    for d in range(3):
        s[d] = np.eye(32, k=1 - d, dtype=np.float32)
    return s


_S_NP = _np_band_sel()
_MAVG_NP = (np.kron(np.eye(32, dtype=np.float32),
                    np.ones((32, 1), np.float32)) / 32.0)


def _np_ef():
    ef = np.zeros((48, 48), np.float32)
    for r in range(48):
        i, k = r // 8, r % 8
        if i < 3:
            ef[r, 6 * k + i] = 1.0
        else:
            ef[r, 6 * k + 3 + (i - 3)] = 1.0
    return ef


_EF_NP = _np_ef()


def _np_selfconj():
    z = np.zeros((_H, _KW), np.float32)
    for r in (0, _H // 2):
        for c in (0, _W // 2):
            z[r, c] = 1.0
    return z


_SC_NP = _np_selfconj()


def _spec_body(x_ref, cw_ref, sw_ref, cr_ref, ci_ref, u_ref, sc_ref,
               mag_ref, msp_ref, psp_ref):
    inv_pi = np.float32(1.0 / np.pi)
    scb = sc_ref[...] > 0.5
    for c in range(3):
        xc = x_ref[0, c]
        xr = jnp.dot(xc, cw_ref[...], preferred_element_type=F32)
        xi = jnp.dot(xc, sw_ref[...], preferred_element_type=F32)
        p = jnp.concatenate([xr, xi], axis=0)
        yr = jnp.dot(cr_ref[...], p, preferred_element_type=F32)
        yi = jnp.dot(ci_ref[...], p, preferred_element_type=F32)
        yi = jnp.where(scb, 0.0, yi)
        m = jnp.clip(jnp.sqrt(yr * yr + yi * yi), 1e-8, 1e6)
        mag_ref[0, c] = m
        ml = jnp.clip(jnp.log1p(m), -20.0, 20.0)
        ph = jnp.clip(jnp.arctan2(yi, yr) * inv_pi, -1.0, 1.0)
        msp_ref[0, c] = jnp.dot(ml, u_ref[...], preferred_element_type=F32)
        psp_ref[0, c] = jnp.dot(ph, u_ref[...], preferred_element_type=F32)


def _conv_body(x0_ref, w1_ref, w2_ref, s1_ref, t1_ref, s2_ref, t2_ref,
               mask_ref, out_ref, x1_ref):
    x1_ref[:, 0:_MARG] = jnp.zeros((96, _MARG), F32)
    x1_ref[:, _MARG + _SPAN:_X1W] = jnp.zeros((96, _X1W - _MARG - _SPAN), F32)
    w1c = w1_ref[...]
    w2c = w2_ref[...]
    s1 = s1_ref[...]
    t1 = t1_ref[...]
    s2 = s2_ref[...]
    t2 = t2_ref[...]

    def body1(i, c):
        bw = pl.multiple_of(128 + i * _CH, 128)
        xw = x0_ref[0, :, pl.ds(bw, _WIN)]
        p = jnp.concatenate(
            [xw[:, 384 + off:384 + off + _CH] for off in _OFFS], axis=0)
        acc = jnp.dot(w1c, p, preferred_element_type=F32)
        v = jnp.maximum(acc * s1 + t1, 0.0)
        v = v * mask_ref[:, pl.ds(i * _CH, _CH)]
        x1_ref[:, pl.ds(pl.multiple_of(_MARG + i * _CH, 128), _CH)] = v
        return c

    lax.fori_loop(0, _NCH, body1, 0, unroll=16)

    def body2(i, pool):
        bw = pl.multiple_of(128 + i * _CH, 128)
        xw = x1_ref[:, pl.ds(bw, _WIN)]
        p = jnp.concatenate(
            [xw[:, 384 + off:384 + off + _CH] for off in _OFFS], axis=0)
        acc = jnp.dot(w2c, p, preferred_element_type=F32)
        v = jnp.maximum(acc * s2 + t2, 0.0)
        v = v * mask_ref[:, pl.ds(i * _CH, _CH)]
        return pool + v

    pool = lax.fori_loop(0, _NCH, body2, jnp.zeros((96, _CH), F32), unroll=16)
    out_ref[0] = jnp.sum(pool, axis=1, keepdims=True) * np.float32(1.0 / 65536.0)


def _head_body(mag_ref, binb_ref, pool_ref, w1b_ref, w2b_ref, mavg_ref,
               ef_ref, rs1_ref, rt1_ref, rs2_ref, rt2_ref, rci_ref,
               aci_ref, acn_ref, ac1_ref, lw1_ref, lb1_ref, lng_ref,
               lnb_ref, lw2_ref, lb2_ref, out_ref):
    mag = mag_ref[...]
    mag96 = jnp.concatenate([mag, mag * mag], axis=0)
    d = jnp.dot(mag96, binb_ref[...], preferred_element_type=F32)
    rad = jnp.clip(d[0:48, 0:32] * rci_ref[...], 0.0, 1e6)
    rad16 = jnp.concatenate([rad[0:16], rad[16:32], rad[32:48]], axis=1)
    y1 = jnp.maximum(jnp.dot(rad16, w1b_ref[...],
                             preferred_element_type=F32)
                     * rs1_ref[...] + rt1_ref[...], 0.0)
    y2 = jnp.maximum(jnp.dot(y1, w2b_ref[...],
                             preferred_element_type=F32)
                     * rs2_ref[...] + rt2_ref[...], 0.0)
    rf = jnp.dot(y2, mavg_ref[...], preferred_element_type=F32)
    s1 = d[0:48, 32:40] * aci_ref[...]
    var = (d[48:96, 32:40] - acn_ref[...] * s1 * s1) * ac1_ref[...]
    std = jnp.sqrt(jnp.maximum(var, 0.0))
    ms = jnp.concatenate([s1[0:16], s1[16:32], s1[32:48],
                          std[0:16], std[16:32], std[32:48]], axis=1)
    az = jnp.dot(ms, ef_ref[...], preferred_element_type=F32)
    comb = jnp.concatenate([pool_ref[...], rf, az], axis=1)
    h = jnp.dot(comb, lw1_ref[...], preferred_element_type=F32) + lb1_ref[...]
    mu = jnp.mean(h, axis=-1, keepdims=True)
    dv = h - mu
    v2 = jnp.mean(dv * dv, axis=-1, keepdims=True)
    h = dv * lax.rsqrt(v2 + 1e-5) * lng_ref[...] + lnb_ref[...]
    h = jnp.maximum(h, 0.0)
    o = jnp.dot(h, lw2_ref[...], preferred_element_type=F32) + lb2_ref[...]
    out_ref[...] = jnp.clip(o, -100.0, 100.0)


def _bn_fold(bias, bnp):
    g, b, m, v = bnp
    s = g * lax.rsqrt(v + 1e-5)
    return s, (bias - m) * s + b


_CPARAMS = dict(dimension_semantics=("parallel", "arbitrary"),
                vmem_limit_bytes=56 * 1024 * 1024)


def kernel(x, mw1, mb1, mbn1, mw2, mb2, mbn2, pw1, pb1, pbn1, pw2, pb2, pbn2,
           rw1, rb1, rbn1, rw2, rb2, rbn2, lw1, lb1, lng, lnb, lw2, lb2):
    B = x.shape[0]

    cw = jnp.asarray(_CW_NP)
    sw = jnp.asarray(_SW_NP)
    cr = jnp.asarray(_CR_NP)
    ci = jnp.asarray(_CI_NP)
    u = jax.image.resize(jnp.eye(_KW, dtype=F32), (_KW, _W), 'bilinear')
    fy = jnp.fft.fftfreq(_H).astype(F32)
    fx = jnp.fft.rfftfreq(_W).astype(F32)
    fyg, fxg = jnp.meshgrid(fy, fx, indexing='ij')
    radius = jnp.sqrt(fxg ** 2 + fyg ** 2)
    rbins = jnp.clip((radius / (radius.max() + 1e-8) * 31).astype(jnp.int32),
                     0, 31).reshape(-1)
    angle = jnp.arctan2(fyg, fxg + 1e-8)
    abins = jnp.clip(((angle + np.pi) / (2 * np.pi) * 8).astype(jnp.int32),
                     0, 7).reshape(-1)
    r_oh = jax.nn.one_hot(rbins, 32, dtype=F32)
    a_oh = jax.nn.one_hot(abins, 8, dtype=F32)
    binb = jnp.concatenate([r_oh, a_oh], axis=1)
    rci = (1.0 / jnp.maximum(r_oh.sum(0), 1.0)).reshape(1, 32)
    a_cnt = a_oh.sum(0)
    aci = jnp.where(a_cnt > 0, 1.0 / a_cnt, 0.0).reshape(1, 8)
    acn = a_cnt.reshape(1, 8)
    ac1 = (1.0 / jnp.maximum(a_cnt - 1.0, 1.0)).reshape(1, 8)
    mask = jnp.asarray(_MASK_NP)
    ef = jnp.asarray(_EF_NP)
    mavg = jnp.asarray(_MAVG_NP)

    xc = jnp.clip(x.astype(F32), -10.0, 10.0)
    w1 = jnp.zeros((9, 96, 6), F32)
    w1 = w1.at[:, :64, :3].set(mw1.transpose(2, 3, 0, 1).reshape(9, 64, 3))
    w1 = w1.at[:, 64:, 3:].set(pw1.transpose(2, 3, 0, 1).reshape(9, 32, 3))
    w1cat = w1.transpose(1, 0, 2).reshape(96, 54)
    w2 = jnp.zeros((9, 96, 96), F32)
    w2 = w2.at[:, :64, :64].set(mw2.transpose(2, 3, 0, 1).reshape(9, 64, 64))
    w2 = w2.at[:, 64:, 64:].set(pw2.transpose(2, 3, 0, 1).reshape(9, 32, 32))
    w2cat = w2.transpose(1, 0, 2).reshape(96, 864)
    ms1, mt1 = _bn_fold(mb1, mbn1)
    ps1, pt1 = _bn_fold(pb1, pbn1)
    s1v = jnp.concatenate([ms1, ps1]).reshape(96, 1)
    t1v = jnp.concatenate([mt1, pt1]).reshape(96, 1)
    ms2, mt2 = _bn_fold(mb2, mbn2)
    ps2, pt2 = _bn_fold(pb2, pbn2)
    s2v = jnp.concatenate([ms2, ps2]).reshape(96, 1)
    t2v = jnp.concatenate([mt2, pt2]).reshape(96, 1)
    sband = jnp.asarray(_S_NP)
    w1b = jnp.einsum('ocd,dut->cuot', rw1, sband).reshape(96, 1024)
    w2b = jnp.einsum('ocd,dut->cuot', rw2, sband).reshape(1024, 1024)
    rs1_, rt1_ = _bn_fold(rb1, rbn1)
    rs2_, rt2_ = _bn_fold(rb2, rbn2)
    rs1 = jnp.repeat(rs1_, 32).reshape(1, 1024)
    rt1 = jnp.repeat(rt1_, 32).reshape(1, 1024)
    rs2 = jnp.repeat(rs2_, 32).reshape(1, 1024)
    rt2 = jnp.repeat(rt2_, 32).reshape(1, 1024)

    bcast2 = lambda c, j: (0, 0)
    bsel = lambda c, j: (c * (B // 2) + j, 0, 0, 0)
    mag, msp, psp = pl.pallas_call(
        _spec_body,
        grid=(2, B // 2),
        in_specs=[
            pl.BlockSpec((1, 3, _H, _W), bsel),
            pl.BlockSpec((_W, _KW), bcast2),
            pl.BlockSpec((_W, _KW), bcast2),
            pl.BlockSpec((_H, 2 * _H), bcast2),
            pl.BlockSpec((_H, 2 * _H), bcast2),
            pl.BlockSpec((_KW, _W), bcast2),
            pl.BlockSpec((_H, _KW), bcast2),
        ],
        out_specs=[
            pl.BlockSpec((1, 3, _H, _KW), bsel),
            pl.BlockSpec((1, 3, _H, _W), bsel),
            pl.BlockSpec((1, 3, _H, _W), bsel),
        ],
        out_shape=[
            jax.ShapeDtypeStruct((B, 3, _H, _KW), F32),
            jax.ShapeDtypeStruct((B, 3, _H, _W), F32),
            jax.ShapeDtypeStruct((B, 3, _H, _W), F32),
        ],
        compiler_params=pltpu.CompilerParams(**_CPARAMS),
        name="spectral_fft",
    )(xc, cw, sw, cr, ci, u, jnp.asarray(_SC_NP))

    sp = jnp.concatenate([msp, psp], axis=1)
    spp = jnp.pad(sp, ((0, 0), (0, 0), (1, 1), (1, 1)))
    x0 = spp.reshape(B, 6, _LP)
    x0e = jnp.pad(x0, ((0, 0), (0, 0), (_MARG, _X0W - _MARG - _LP)))
    mag48 = mag.transpose(1, 0, 2, 3).reshape(3 * B, _NF)

    bsel3 = lambda c, j: (c * (B // 2) + j, 0, 0)
    pool = pl.pallas_call(
        _conv_body,
        grid=(2, B // 2),
        in_specs=[
            pl.BlockSpec((1, 6, _X0W), bsel3),
            pl.BlockSpec((96, 54), bcast2),
            pl.BlockSpec((96, 864), bcast2),
            pl.BlockSpec((96, 1), bcast2),
            pl.BlockSpec((96, 1), bcast2),
            pl.BlockSpec((96, 1), bcast2),
            pl.BlockSpec((96, 1), bcast2),
            pl.BlockSpec((1, _SPAN), bcast2),
        ],
        out_specs=pl.BlockSpec((1, 96, 1), bsel3),
        out_shape=jax.ShapeDtypeStruct((B, 96, 1), F32),
        scratch_shapes=[pltpu.VMEM((96, _X1W), F32)],
        compiler_params=pltpu.CompilerParams(**_CPARAMS),
        name="conv_branches",
    )(x0e, w1cat, w2cat, s1v, t1v, s2v, t2v, mask)
    poolt = pool.reshape(B, 96)

    out = pl.pallas_call(
        _head_body,
        out_shape=jax.ShapeDtypeStruct((B, 128), F32),
        compiler_params=pltpu.CompilerParams(
            vmem_limit_bytes=56 * 1024 * 1024),
        name="bin_head",
    )(mag48, binb, poolt, w1b, w2b, mavg, ef, rs1, rt1, rs2, rt2,
      rci, aci, acn, ac1, lw1.T, lb1.reshape(1, -1), lng.reshape(1, -1),
      lnb.reshape(1, -1), lw2.T, lb2.reshape(1, -1))
    return out


# speedup vs baseline: 2.1920x; 1.0006x over previous
import numpy as np

import jax
import jax.numpy as jnp
from jax import lax
from jax.experimental import pallas as pl
from jax.experimental.pallas import tpu as pltpu

F32 = jnp.float32

_H = 256
_W = 256
_KW = 129
_NF = _H * _KW
_WP = 258
_LP = _WP * _WP
_MARG = 512
_CH = 512
_NCH = 131
_SPAN = _CH * _NCH
_X0W = 67968
_X1W = 67968
_OFFS = [dy * _WP + dx for dy in (-1, 0, 1) for dx in (-1, 0, 1)]
_WIN = 1280


def _np_dft_consts():
    w = np.arange(_W)[:, None].astype(np.float64)
    k = np.arange(_KW)[None, :].astype(np.float64)
    ang = 2.0 * np.pi / _W * (w * k)
    cw = np.cos(ang).astype(np.float32)
    sw = (-np.sin(ang)).astype(np.float32)
    ky = np.arange(_H)[:, None].astype(np.float64)
    h = np.arange(_H)[None, :].astype(np.float64)
    ang2 = 2.0 * np.pi / _H * (ky * h)
    ch = np.cos(ang2).astype(np.float32)
    sh = np.sin(ang2).astype(np.float32)
    cr = np.concatenate([ch, sh], axis=1)
    ci = np.concatenate([-sh, ch], axis=1)
    return cw, sw, cr, ci


_CW_NP, _SW_NP, _CR_NP, _CI_NP = _np_dft_consts()


def _np_mask():
    p = np.arange(_SPAN)
    row = p // _WP
    col = p % _WP
    valid = ((row >= 1) & (row <= _W) & (col >= 1) & (col <= _W)
             & (p < _LP))
    return valid.astype(np.float32)[None, :]


_MASK_NP = _np_mask()


def _np_band_sel():
    s = np.zeros((3, 32, 32), np.float32)
    for d in range(3):
        s[d] = np.eye(32, k=1 - d, dtype=np.float32)
    return s


_S_NP = _np_band_sel()
_MAVG_NP = (np.kron(np.eye(32, dtype=np.float32),
                    np.ones((32, 1), np.float32)) / 32.0)


def _np_ef():
    ef = np.zeros((48, 48), np.float32)
    for r in range(48):
        i, k = r // 8, r % 8
        if i < 3:
            ef[r, 6 * k + i] = 1.0
        else:
            ef[r, 6 * k + 3 + (i - 3)] = 1.0
    return ef


_EF_NP = _np_ef()


def _np_selfconj():
    z = np.zeros((_H, _KW), np.float32)
    for r in (0, _H // 2):
        for c in (0, _W // 2):
            z[r, c] = 1.0
    return z


_SC_NP = _np_selfconj()


def _spec_body(x_ref, cw_ref, sw_ref, cr_ref, ci_ref, u_ref, sc_ref,
               mag_ref, msp_ref, psp_ref):
    inv_pi = np.float32(1.0 / np.pi)
    scb = sc_ref[...] > 0.5
    for c in range(3):
        xc = x_ref[0, c]
        xr = jnp.dot(xc, cw_ref[...], preferred_element_type=F32)
        xi = jnp.dot(xc, sw_ref[...], preferred_element_type=F32)
        p = jnp.concatenate([xr, xi], axis=0)
        yr = jnp.dot(cr_ref[...], p, preferred_element_type=F32)
        yi = jnp.dot(ci_ref[...], p, preferred_element_type=F32)
        yi = jnp.where(scb, 0.0, yi)
        m = jnp.clip(jnp.sqrt(yr * yr + yi * yi), 1e-8, 1e6)
        mag_ref[0, c] = m
        ml = jnp.clip(jnp.log1p(m), -20.0, 20.0)
        ph = jnp.clip(jnp.arctan2(yi, yr) * inv_pi, -1.0, 1.0)
        msp_ref[0, c] = jnp.dot(ml, u_ref[...], preferred_element_type=F32)
        psp_ref[0, c] = jnp.dot(ph, u_ref[...], preferred_element_type=F32)


def _conv_body(x0_ref, w1_ref, w2_ref, s1_ref, t1_ref, s2_ref, t2_ref,
               mask_ref, out_ref, x1_ref):
    x1_ref[:, 0:_MARG] = jnp.zeros((96, _MARG), F32)
    x1_ref[:, _MARG + _SPAN:_X1W] = jnp.zeros((96, _X1W - _MARG - _SPAN), F32)
    w1c = w1_ref[...]
    w2c = w2_ref[...]
    s1 = s1_ref[...]
    t1 = t1_ref[...]
    s2 = s2_ref[...]
    t2 = t2_ref[...]

    def body1(i, c):
        bw = pl.multiple_of(128 + i * _CH, 128)
        xw = x0_ref[0, :, pl.ds(bw, _WIN)]
        p = jnp.concatenate(
            [xw[:, 384 + off:384 + off + _CH] for off in _OFFS], axis=0)
        acc = jnp.dot(w1c, p, preferred_element_type=F32)
        v = jnp.maximum(acc * s1 + t1, 0.0)
        v = v * mask_ref[:, pl.ds(i * _CH, _CH)]
        x1_ref[:, pl.ds(pl.multiple_of(_MARG + i * _CH, 128), _CH)] = v
        return c

    lax.fori_loop(0, _NCH, body1, 0, unroll=16)

    def body2(i, pool):
        bw = pl.multiple_of(128 + i * _CH, 128)
        xw = x1_ref[:, pl.ds(bw, _WIN)]
        p = jnp.concatenate(
            [xw[:, 384 + off:384 + off + _CH] for off in _OFFS], axis=0)
        acc = jnp.dot(w2c, p, preferred_element_type=F32)
        v = jnp.maximum(acc * s2 + t2, 0.0)
        v = v * mask_ref[:, pl.ds(i * _CH, _CH)]
        return pool + v

    pool = lax.fori_loop(0, _NCH, body2, jnp.zeros((96, _CH), F32), unroll=16)
    out_ref[0] = jnp.sum(pool, axis=1, keepdims=True) * np.float32(1.0 / 65536.0)


def _head_body(mag_ref, binb_ref, pool_ref, w1b_ref, w2b_ref, mavg_ref,
               ef_ref, rs1_ref, rt1_ref, rs2_ref, rt2_ref, rci_ref,
               aci_ref, acn_ref, ac1_ref, lw1_ref, lb1_ref, lng_ref,
               lnb_ref, lw2_ref, lb2_ref, out_ref):
    mag = mag_ref[...]
    mag96 = jnp.concatenate([mag, mag * mag], axis=0)
    d = jnp.dot(mag96, binb_ref[...], preferred_element_type=F32)
    rad = jnp.clip(d[0:48, 0:32] * rci_ref[...], 0.0, 1e6)
    rad16 = jnp.concatenate([rad[0:16], rad[16:32], rad[32:48]], axis=1)
    y1 = jnp.maximum(jnp.dot(rad16, w1b_ref[...],
                             preferred_element_type=F32)
                     * rs1_ref[...] + rt1_ref[...], 0.0)
    y2 = jnp.maximum(jnp.dot(y1, w2b_ref[...],
                             preferred_element_type=F32)
                     * rs2_ref[...] + rt2_ref[...], 0.0)
    rf = jnp.dot(y2, mavg_ref[...], preferred_element_type=F32)
    s1 = d[0:48, 32:40] * aci_ref[...]
    var = (d[48:96, 32:40] - acn_ref[...] * s1 * s1) * ac1_ref[...]
    std = jnp.sqrt(jnp.maximum(var, 0.0))
    ms = jnp.concatenate([s1[0:16], s1[16:32], s1[32:48],
                          std[0:16], std[16:32], std[32:48]], axis=1)
    az = jnp.dot(ms, ef_ref[...], preferred_element_type=F32)
    comb = jnp.concatenate([pool_ref[...], rf, az], axis=1)
    h = jnp.dot(comb, lw1_ref[...], preferred_element_type=F32) + lb1_ref[...]
    mu = jnp.mean(h, axis=-1, keepdims=True)
    dv = h - mu
    v2 = jnp.mean(dv * dv, axis=-1, keepdims=True)
    h = dv * lax.rsqrt(v2 + 1e-5) * lng_ref[...] + lnb_ref[...]
    h = jnp.maximum(h, 0.0)
    o = jnp.dot(h, lw2_ref[...], preferred_element_type=F32) + lb2_ref[...]
    out_ref[...] = jnp.clip(o, -100.0, 100.0)


def _bn_fold(bias, bnp):
    g, b, m, v = bnp
    s = g * lax.rsqrt(v + 1e-5)
    return s, (bias - m) * s + b


_CPARAMS = dict(dimension_semantics=("parallel", "arbitrary"),
                vmem_limit_bytes=56 * 1024 * 1024)


def kernel(x, mw1, mb1, mbn1, mw2, mb2, mbn2, pw1, pb1, pbn1, pw2, pb2, pbn2,
           rw1, rb1, rbn1, rw2, rb2, rbn2, lw1, lb1, lng, lnb, lw2, lb2):
    B = x.shape[0]

    cw = jnp.asarray(_CW_NP)
    sw = jnp.asarray(_SW_NP)
    cr = jnp.asarray(_CR_NP)
    ci = jnp.asarray(_CI_NP)
    u = jax.image.resize(jnp.eye(_KW, dtype=F32), (_KW, _W), 'bilinear')
    fy = jnp.fft.fftfreq(_H).astype(F32)
    fx = jnp.fft.rfftfreq(_W).astype(F32)
    fyg, fxg = jnp.meshgrid(fy, fx, indexing='ij')
    radius = jnp.sqrt(fxg ** 2 + fyg ** 2)
    rbins = jnp.clip((radius / (radius.max() + 1e-8) * 31).astype(jnp.int32),
                     0, 31).reshape(-1)
    angle = jnp.arctan2(fyg, fxg + 1e-8)
    abins = jnp.clip(((angle + np.pi) / (2 * np.pi) * 8).astype(jnp.int32),
                     0, 7).reshape(-1)
    r_oh = jax.nn.one_hot(rbins, 32, dtype=F32)
    a_oh = jax.nn.one_hot(abins, 8, dtype=F32)
    binb = jnp.concatenate([r_oh, a_oh], axis=1)
    rci = (1.0 / jnp.maximum(r_oh.sum(0), 1.0)).reshape(1, 32)
    a_cnt = a_oh.sum(0)
    aci = jnp.where(a_cnt > 0, 1.0 / a_cnt, 0.0).reshape(1, 8)
    acn = a_cnt.reshape(1, 8)
    ac1 = (1.0 / jnp.maximum(a_cnt - 1.0, 1.0)).reshape(1, 8)
    mask = jnp.asarray(_MASK_NP)
    ef = jnp.asarray(_EF_NP)
    mavg = jnp.asarray(_MAVG_NP)

    xc = jnp.clip(x.astype(F32), -10.0, 10.0)
    w1 = jnp.zeros((9, 96, 6), F32)
    w1 = w1.at[:, :64, :3].set(mw1.transpose(2, 3, 0, 1).reshape(9, 64, 3))
    w1 = w1.at[:, 64:, 3:].set(pw1.transpose(2, 3, 0, 1).reshape(9, 32, 3))
    w1cat = w1.transpose(1, 0, 2).reshape(96, 54)
    w2 = jnp.zeros((9, 96, 96), F32)
    w2 = w2.at[:, :64, :64].set(mw2.transpose(2, 3, 0, 1).reshape(9, 64, 64))
    w2 = w2.at[:, 64:, 64:].set(pw2.transpose(2, 3, 0, 1).reshape(9, 32, 32))
    w2cat = w2.transpose(1, 0, 2).reshape(96, 864)
    ms1, mt1 = _bn_fold(mb1, mbn1)
    ps1, pt1 = _bn_fold(pb1, pbn1)
    s1v = jnp.concatenate([ms1, ps1]).reshape(96, 1)
    t1v = jnp.concatenate([mt1, pt1]).reshape(96, 1)
    ms2, mt2 = _bn_fold(mb2, mbn2)
    ps2, pt2 = _bn_fold(pb2, pbn2)
    s2v = jnp.concatenate([ms2, ps2]).reshape(96, 1)
    t2v = jnp.concatenate([mt2, pt2]).reshape(96, 1)
    sband = jnp.asarray(_S_NP)
    w1b = jnp.einsum('ocd,dut->cuot', rw1, sband).reshape(96, 1024)
    w2b = jnp.einsum('ocd,dut->cuot', rw2, sband).reshape(1024, 1024)
    rs1_, rt1_ = _bn_fold(rb1, rbn1)
    rs2_, rt2_ = _bn_fold(rb2, rbn2)
    rs1 = jnp.repeat(rs1_, 32).reshape(1, 1024)
    rt1 = jnp.repeat(rt1_, 32).reshape(1, 1024)
    rs2 = jnp.repeat(rs2_, 32).reshape(1, 1024)
    rt2 = jnp.repeat(rt2_, 32).reshape(1, 1024)

    bcast2 = lambda c, j: (0, 0)
    bsel = lambda c, j: (c * (B // 2) + j, 0, 0, 0)
    mag, msp, psp = pl.pallas_call(
        _spec_body,
        grid=(2, B // 2),
        in_specs=[
            pl.BlockSpec((1, 3, _H, _W), bsel),
            pl.BlockSpec((_W, _KW), bcast2),
            pl.BlockSpec((_W, _KW), bcast2),
            pl.BlockSpec((_H, 2 * _H), bcast2),
            pl.BlockSpec((_H, 2 * _H), bcast2),
            pl.BlockSpec((_KW, _W), bcast2),
            pl.BlockSpec((_H, _KW), bcast2),
        ],
        out_specs=[
            pl.BlockSpec((1, 3, _H, _KW), bsel),
            pl.BlockSpec((1, 3, _H, _W), bsel),
            pl.BlockSpec((1, 3, _H, _W), bsel),
        ],
        out_shape=[
            jax.ShapeDtypeStruct((B, 3, _H, _KW), F32),
            jax.ShapeDtypeStruct((B, 3, _H, _W), F32),
            jax.ShapeDtypeStruct((B, 3, _H, _W), F32),
        ],
        compiler_params=pltpu.CompilerParams(**_CPARAMS),
        name="spectral_fft",
    )(xc, cw, sw, cr, ci, u, jnp.asarray(_SC_NP))

    sp = jnp.concatenate([msp, psp], axis=1)
    spp = jnp.pad(sp, ((0, 0), (0, 0), (1, 1), (1, 1)))
    x0 = spp.reshape(B, 6, _LP)
    x0e = jnp.pad(x0, ((0, 0), (0, 0), (_MARG, _X0W - _MARG - _LP)))
    mag48 = mag.transpose(1, 0, 2, 3).reshape(3 * B, _NF)

    bsel3 = lambda c, j: (c * (B // 2) + j, 0, 0)
    pool = pl.pallas_call(
        _conv_body,
        grid=(2, B // 2),
        in_specs=[
            pl.BlockSpec((1, 6, _X0W), bsel3),
            pl.BlockSpec((96, 54), bcast2),
            pl.BlockSpec((96, 864), bcast2),
            pl.BlockSpec((96, 1), bcast2),
            pl.BlockSpec((96, 1), bcast2),
            pl.BlockSpec((96, 1), bcast2),
            pl.BlockSpec((96, 1), bcast2),
            pl.BlockSpec((1, _SPAN), bcast2),
        ],
        out_specs=pl.BlockSpec((1, 96, 1), bsel3),
        out_shape=jax.ShapeDtypeStruct((B, 96, 1), F32),
        scratch_shapes=[pltpu.VMEM((96, _X1W), F32)],
        compiler_params=pltpu.CompilerParams(**_CPARAMS),
        name="conv_branches",
    )(x0e, w1cat, w2cat, s1v, t1v, s2v, t2v, mask)
    poolt = pool.reshape(B, 96)

    out = pl.pallas_call(
        _head_body,
        out_shape=jax.ShapeDtypeStruct((B, 128), F32),
        compiler_params=pltpu.CompilerParams(
            vmem_limit_bytes=56 * 1024 * 1024),
        name="bin_head",
    )(mag48, binb, poolt, w1b, w2b, mavg, ef, rs1, rt1, rs2, rt2,
      rci, aci, acn, ac1, lw1.T, lb1.reshape(1, -1), lng.reshape(1, -1),
      lnb.reshape(1, -1), lw2.T, lb2.reshape(1, -1))
    return out
